# Initial kernel scaffold; baseline (speedup 1.0000x reference)
import jax
import jax.numpy as jnp
from jax.experimental import pallas as pl


def kernel(x, emb_weight):
    raise NotImplementedError("write your pallas kernel here")


# speedup vs baseline: 1.3163x; 1.3163x over previous
import functools

import jax
import jax.numpy as jnp
from jax import lax
from jax.experimental import pallas as pl
from jax.experimental.pallas import tpu as pltpu
from jax.experimental.pallas import tpu_sc as plsc

N = 100000
HIDDEN = 256
NUM_EMB = 95

NC = 2
NS = 16
NW = NC * NS

CHUNK = 80
NCHUNKS = N // CHUNK


def _swish_table(w):
    def body(w_ref, o_ref):
        v = w_ref[...]
        o_ref[...] = v * (1.0 / (1.0 + jnp.exp(-v)))
    return pl.pallas_call(
        body,
        out_shape=jax.ShapeDtypeStruct(w.shape, w.dtype),
    )(w)


def _make_sc_gather():
    mesh = plsc.VectorSubcoreMesh(core_axis_name="c", subcore_axis_name="s")

    @functools.partial(
        pl.kernel,
        mesh=mesh,
        out_type=jax.ShapeDtypeStruct((N, HIDDEN), jnp.float32),
        scratch_types=[
            pltpu.VMEM((CHUNK,), jnp.int32),
            pltpu.VMEM((CHUNK, HIDDEN), jnp.float32),
            pltpu.SemaphoreType.DMA,
        ],
    )
    def sc_gather(table_hbm, idx_hbm, out_hbm, idx_v, rows_v, sem):
        w = lax.axis_index("s") * NC + lax.axis_index("c")
        nchunks_w = (NCHUNKS - w + NW - 1) // NW

        def body(i, carry):
            c = w + i * NW
            base = pl.multiple_of(c * CHUNK, 8)
            pltpu.sync_copy(idx_hbm.at[pl.ds(base, CHUNK)], idx_v)
            pltpu.async_copy(table_hbm.at[idx_v], rows_v, sem).wait()
            pltpu.sync_copy(rows_v, out_hbm.at[pl.ds(base, CHUNK)])
            return carry

        lax.fori_loop(0, nchunks_w, body, 0)

    return sc_gather


_sc_gather = _make_sc_gather()


def kernel(x, emb_weight):
    act_table = _swish_table(emb_weight)
    return _sc_gather(act_table, x.astype(jnp.int32))
